# Initial kernel scaffold; baseline (speedup 1.0000x reference)
import jax
import jax.numpy as jnp
from jax.experimental import pallas as pl


def kernel(chn_llr, msg_C2V, msg_V2C, gamma, edge_var, edge_chk, Wi, We):
    raise NotImplementedError("write your pallas kernel here")


# speedup vs baseline: 2.3614x; 2.3614x over previous
import functools

import jax
import jax.numpy as jnp
import numpy as np
from jax import lax
from jax.experimental import pallas as pl
from jax.experimental.pallas import tpu as pltpu
from jax.experimental.pallas import tpu_sc as plsc

N = 50000
M = 25000
E = 150000
B = 128
LLR_CLIP = 15.0
LB = float(-np.log(np.tanh(LLR_CLIP / 2)))

NC, NS, L = 2, 16, 16
K = 120
NCH = E // K
WR = 200

_mesh = plsc.VectorSubcoreMesh(core_axis_name="c", subcore_axis_name="s")


def _halves(buf, r):
    return (buf[r, pl.ds(0, 16)], buf[r, pl.ds(16, 16)])


@functools.partial(
    pl.kernel,
    out_type=jax.ShapeDtypeStruct((N, B), jnp.float32),
    mesh=_mesh,
    compiler_params=pltpu.CompilerParams(use_tc_tiling_on_sc=False, needs_layout_passes=False),
    scratch_types=[
        pltpu.VMEM_SHARED((N, 32), jnp.float32),
        pltpu.VMEM((WR, 32), jnp.float32),
        pltpu.VMEM((WR, 32), jnp.float32),
        pltpu.VMEM((K, 32), jnp.float32),
        pltpu.VMEM((K,), jnp.int32),
        pltpu.VMEM((2, 16), jnp.float32),
    ],
)
def _seg_affine(vals, idx, chn, par, out, acc, sbuf, cbuf, vbuf, ibuf, pbuf):
    c = lax.axis_index("c")
    s = lax.axis_index("s")
    pltpu.sync_copy(par, pbuf)
    zv = jnp.zeros((16,), jnp.float32)
    nzch = N // WR
    kz = (nzch + NS - 1) // NS

    for qj in range(2):
        co = 32 * (2 * c + qj)
        def zrow(r, _):
            sbuf[r, pl.ds(0, 16)] = zv
            sbuf[r, pl.ds(16, 16)] = zv
            return _
        lax.fori_loop(0, WR, zrow, None)
        def zchunk(kk, _):
            j = s + NS * kk
            @pl.when(j < nzch)
            def _():
                pltpu.sync_copy(sbuf, acc.at[pl.ds(j * WR, WR)])
            return _
        lax.fori_loop(0, kz, zchunk, None)
        plsc.subcore_barrier()

        def accum(kk, _):
            j = s + NS * kk
            @pl.when(j < NCH)
            def _():
                e0 = j * K
                pltpu.sync_copy(idx.at[pl.ds(e0, K)], ibuf)
                pltpu.sync_copy(vals.at[pl.ds(e0, K), pl.ds(co, 32)], vbuf)
                pltpu.sync_copy(vbuf, acc.at[ibuf], add=True)
            return _
        lax.fori_loop(0, (NCH + NS - 1) // NS, accum, None)
        plsc.subcore_barrier()

        wi = pbuf[0]
        we = pbuf[1]
        def wchunk(kk, _):
            j = s + NS * kk
            @pl.when(j < nzch)
            def _():
                r0 = j * WR
                pltpu.sync_copy(acc.at[pl.ds(r0, WR)], sbuf)
                pltpu.sync_copy(chn.at[pl.ds(r0, WR), pl.ds(co, 32)], cbuf)
                def wrow(r, _):
                    for h in (0, 16):
                        sl = pl.ds(h, 16)
                        sbuf[r, sl] = wi * cbuf[r, sl] + we * sbuf[r, sl]
                    return _
                lax.fori_loop(0, WR, wrow, None)
                pltpu.sync_copy(sbuf, out.at[pl.ds(r0, WR), pl.ds(co, 32)])
            return _
        lax.fori_loop(0, kz, wchunk, None)
        plsc.subcore_barrier()


@functools.partial(
    pl.kernel,
    out_type=jax.ShapeDtypeStruct((E, B), jnp.float32),
    mesh=_mesh,
    compiler_params=pltpu.CompilerParams(use_tc_tiling_on_sc=False, needs_layout_passes=False),
    scratch_types=[
        pltpu.VMEM((K, B), jnp.float32),
        pltpu.VMEM((K, B), jnp.float32),
        pltpu.VMEM((K, B), jnp.float32),
        pltpu.VMEM((K,), jnp.int32),
        pltpu.VMEM((4, 16), jnp.float32),
        pltpu.SemaphoreType.DMA,
        pltpu.SemaphoreType.DMA,
        pltpu.SemaphoreType.DMA,
    ],
)
def _vstep(mv2c, mc2v, ellps, idx, par, out,
           abuf, bbuf, gbuf, ibuf, pbuf, sem1, sem2, sem3):
    c = lax.axis_index("c")
    s = lax.axis_index("s")
    wid = s * NC + c
    pltpu.sync_copy(par, pbuf)
    omg = pbuf[0]
    g = pbuf[1]
    we = pbuf[2]

    def step(kk, _):
        j = wid + NC * NS * kk
        @pl.when(j < NCH)
        def _():
            e0 = j * K
            pltpu.sync_copy(idx.at[pl.ds(e0, K)], ibuf)
            d1 = pltpu.async_copy(ellps.at[ibuf], gbuf, sem1)
            d2 = pltpu.async_copy(mv2c.at[pl.ds(e0, K)], abuf, sem2)
            d3 = pltpu.async_copy(mc2v.at[pl.ds(e0, K)], bbuf, sem3)
            d1.wait()
            d2.wait()
            d3.wait()
            def row(r, _):
                for h in range(8):
                    sl = pl.ds(16 * h, 16)
                    abuf[r, sl] = (omg * abuf[r, sl]
                                   + g * (gbuf[r, sl] - we * bbuf[r, sl]))
                return _
            lax.fori_loop(0, K, row, None)
            pltpu.sync_copy(abuf, out.at[pl.ds(e0, K)])
        return _
    lax.fori_loop(0, (NCH + NC * NS - 1) // (NC * NS), step, None)


@functools.partial(
    pl.kernel,
    out_type=(jax.ShapeDtypeStruct((M, B), jnp.float32),
              jax.ShapeDtypeStruct((M, B), jnp.float32)),
    mesh=_mesh,
    compiler_params=pltpu.CompilerParams(use_tc_tiling_on_sc=False, needs_layout_passes=False),
    scratch_types=[
        pltpu.VMEM_SHARED((M, 32), jnp.float32),
        pltpu.VMEM_SHARED((M, 32), jnp.float32),
        pltpu.VMEM((WR, 32), jnp.float32),
        pltpu.VMEM((K, 32), jnp.float32),
        pltpu.VMEM((K, 32), jnp.float32),
        pltpu.VMEM((K, 32), jnp.float32),
        pltpu.VMEM((K,), jnp.int32),
    ],
)
def _dual_seg(u, idx, st, sn, acct, accn, zbuf, ubuf, tbuf, nbuf, ibuf):
    c = lax.axis_index("c")
    s = lax.axis_index("s")
    zv = jnp.zeros((16,), jnp.float32)
    one = jnp.full((16,), 1.0, jnp.float32)
    nzch = M // WR
    kz = (nzch + NS - 1) // NS

    def zrow(r, _):
        zbuf[r, pl.ds(0, 16)] = zv
        zbuf[r, pl.ds(16, 16)] = zv
        return _
    lax.fori_loop(0, WR, zrow, None)

    for qj in range(2):
        co = 32 * (2 * c + qj)
        def zchunk(kk, _):
            j = s + NS * kk
            @pl.when(j < nzch)
            def _():
                pltpu.sync_copy(zbuf, acct.at[pl.ds(j * WR, WR)])
                pltpu.sync_copy(zbuf, accn.at[pl.ds(j * WR, WR)])
            return _
        lax.fori_loop(0, kz, zchunk, None)
        plsc.subcore_barrier()

        def accum(kk, _):
            j = s + NS * kk
            @pl.when(j < NCH)
            def _():
                e0 = j * K
                pltpu.sync_copy(idx.at[pl.ds(e0, K)], ibuf)
                pltpu.sync_copy(u.at[pl.ds(e0, K), pl.ds(co, 32)], ubuf)
                def row(r, _):
                    for h in (0, 16):
                        sl = pl.ds(h, 16)
                        x = ubuf[r, sl]
                        tbuf[r, sl] = -jnp.abs(x)
                        nbuf[r, sl] = jnp.where(x < 0, one, zv)
                    return _
                lax.fori_loop(0, K, row, None)
                pltpu.sync_copy(tbuf, acct.at[ibuf], add=True)
                pltpu.sync_copy(nbuf, accn.at[ibuf], add=True)
            return _
        lax.fori_loop(0, (NCH + NS - 1) // NS, accum, None)
        plsc.subcore_barrier()

        def wchunk(kk, _):
            j = s + NS * kk
            @pl.when(j < nzch)
            def _():
                r0 = j * WR
                pltpu.sync_copy(acct.at[pl.ds(r0, WR)],
                                st.at[pl.ds(r0, WR), pl.ds(co, 32)])
                pltpu.sync_copy(accn.at[pl.ds(r0, WR)],
                                sn.at[pl.ds(r0, WR), pl.ds(co, 32)])
            return _
        lax.fori_loop(0, kz, wchunk, None)
        plsc.subcore_barrier()


@functools.partial(
    pl.kernel,
    out_type=jax.ShapeDtypeStruct((E, B), jnp.float32),
    mesh=_mesh,
    compiler_params=pltpu.CompilerParams(use_tc_tiling_on_sc=False, needs_layout_passes=False),
    scratch_types=[
        pltpu.VMEM((K, B), jnp.float32),
        pltpu.VMEM((K, B), jnp.float32),
        pltpu.VMEM((K, B), jnp.float32),
        pltpu.VMEM((K,), jnp.int32),
        pltpu.SemaphoreType.DMA,
        pltpu.SemaphoreType.DMA,
        pltpu.SemaphoreType.DMA,
    ],
)
def _hgather(u, st, sn, idx, out, ubuf, tb, nb, ibuf, sem1, sem2, sem3):
    c = lax.axis_index("c")
    s = lax.axis_index("s")
    wid = s * NC + c
    one = jnp.full((16,), 1.0, jnp.float32)
    zv = jnp.zeros((16,), jnp.float32)

    def step(kk, _):
        j = wid + NC * NS * kk
        @pl.when(j < NCH)
        def _():
            e0 = j * K
            pltpu.sync_copy(idx.at[pl.ds(e0, K)], ibuf)
            d1 = pltpu.async_copy(st.at[ibuf], tb, sem1)
            d2 = pltpu.async_copy(sn.at[ibuf], nb, sem2)
            d3 = pltpu.async_copy(u.at[pl.ds(e0, K)], ubuf, sem3)
            d1.wait()
            d2.wait()
            d3.wait()
            def row(r, _):
                for h in range(8):
                    sl = pl.ds(16 * h, 16)
                    uv = ubuf[r, sl]
                    tv = -jnp.abs(uv)
                    negv = jnp.where(uv < 0, one, zv)
                    amp = tb[r, sl] - tv
                    cnt = nb[r, sl] - negv
                    par = lax.shift_left(
                        lax.bitwise_and(cnt.astype(jnp.int32), 1), 31)
                    bits = lax.bitwise_or(
                        plsc.bitcast(jnp.abs(amp), jnp.int32), par)
                    ubuf[r, sl] = plsc.bitcast(bits, jnp.float32)
                return _
            lax.fori_loop(0, K, row, None)
            pltpu.sync_copy(ubuf, out.at[pl.ds(e0, K)])
        return _
    lax.fori_loop(0, (NCH + NC * NS - 1) // (NC * NS), step, None)


_TCR = 1200


def _u_body(x_ref, o_ref):
    x = x_ref[...]
    lam = jnp.clip(x, -LLR_CLIP, LLR_CLIP)
    al = jnp.clip(jnp.abs(lam), LB, LLR_CLIP)
    t = jnp.log(jnp.tanh(al * 0.5))
    o_ref[...] = jnp.where(lam < 0, t, -t)


def _tc_u(x):
    return pl.pallas_call(
        _u_body,
        grid=(E // _TCR,),
        in_specs=[pl.BlockSpec((_TCR, B), lambda i: (i, 0))],
        out_specs=pl.BlockSpec((_TCR, B), lambda i: (i, 0)),
        out_shape=jax.ShapeDtypeStruct((E, B), jnp.float32),
    )(x)


def _h_body(g_ref, w_ref, m_ref, o_ref):
    g = g_ref[0, 0]
    w = w_ref[...]
    wb = lax.bitcast_convert_type(w, jnp.int32)
    sgn = jnp.where(wb < 0, -1.0, 1.0)
    y = jnp.exp(-jnp.abs(w)) * (1.0 - 1e-6)
    at = 0.5 * jnp.log((1.0 + y) / (1.0 - y))
    o_ref[...] = (1.0 - g) * m_ref[...] + g * (sgn * 2.0 * at)


def _tc_h(gamma, w, mc2v):
    return pl.pallas_call(
        _h_body,
        grid=(E // _TCR,),
        in_specs=[
            pl.BlockSpec(memory_space=pltpu.SMEM),
            pl.BlockSpec((_TCR, B), lambda i: (i, 0)),
            pl.BlockSpec((_TCR, B), lambda i: (i, 0)),
        ],
        out_specs=pl.BlockSpec((_TCR, B), lambda i: (i, 0)),
        out_shape=jax.ShapeDtypeStruct((E, B), jnp.float32),
    )(gamma, w, mc2v)


def kernel(chn_llr, msg_C2V, msg_V2C, gamma, edge_var, edge_chk, Wi, We):
    ev = edge_var.astype(jnp.int32)
    ec = edge_chk.astype(jnp.int32)
    wi = Wi[0]
    we = We[0]
    par1 = jnp.stack([jnp.full((16,), wi), jnp.full((16,), we)])
    par2 = jnp.stack([jnp.full((16,), 1.0 - gamma), jnp.full((16,), gamma),
                      jnp.full((16,), we), jnp.zeros((16,))])

    ellps = _seg_affine(msg_C2V, ev, chn_llr, par1)
    mv2c_new = _vstep(msg_V2C, msg_C2V, ellps, ev, par2)
    u = _tc_u(mv2c_new)
    st, sn = _dual_seg(u, ec)
    w = _hgather(u, st, sn, ec)
    mc2v_new = _tc_h(jnp.reshape(gamma, (1, 1)), w, msg_C2V)
    output = _seg_affine(mc2v_new, ev, chn_llr, par1)
    return (mc2v_new, mv2c_new, output)


# speedup vs baseline: 3.4832x; 1.4751x over previous
import functools

import jax
import jax.numpy as jnp
import numpy as np
from jax import lax
from jax.experimental import pallas as pl
from jax.experimental.pallas import tpu as pltpu
from jax.experimental.pallas import tpu_sc as plsc

N = 50000
M = 25000
E = 150000
B = 128
LLR_CLIP = 15.0
LB = float(-np.log(np.tanh(LLR_CLIP / 2)))

NC, NS, L = 2, 16, 16
NW = NC * NS
K = 120
NCH = E // K
CPT = NCH // NS
CPW = 40
WSTART_MAX = NCH - CPW
WR = 200

_mesh = plsc.VectorSubcoreMesh(core_axis_name="c", subcore_axis_name="s")
_params = pltpu.CompilerParams(use_tc_tiling_on_sc=False,
                               needs_layout_passes=False)


@functools.partial(
    pl.kernel,
    out_type=jax.ShapeDtypeStruct((N, B), jnp.float32),
    mesh=_mesh,
    compiler_params=_params,
    scratch_types=[
        pltpu.VMEM_SHARED((N, 32), jnp.float32),
        pltpu.VMEM((WR, 32), jnp.float32),
        pltpu.VMEM((WR, 32), jnp.float32),
        pltpu.VMEM((K, 32), jnp.float32),
        pltpu.VMEM((K, 32), jnp.float32),
        pltpu.VMEM((CPT, K), jnp.int32),
        pltpu.VMEM((1, K), jnp.int32),
        pltpu.VMEM((2, 16), jnp.float32),
        pltpu.SemaphoreType.DMA,
        pltpu.SemaphoreType.DMA,
        pltpu.SemaphoreType.DMA,
    ],
)
def _seg_affine(vals, idx2d, chn, par, out, acc, sbuf, cbuf, vbuf0, vbuf1,
                iball, itail, pbuf, vsem0, vsem1, ssem):
    c = lax.axis_index("c")
    s = lax.axis_index("s")
    pltpu.sync_copy(par, pbuf)
    pltpu.sync_copy(idx2d.at[pl.ds(s * CPT, CPT)], iball)
    zv = jnp.zeros((16,), jnp.float32)
    nzch = N // WR
    kz = (nzch + NS - 1) // NS
    vbufs = (vbuf0, vbuf1)
    vsems = (vsem0, vsem1)

    def _vload(k, b, co):
        e0 = (s * CPT + k) * K
        return pltpu.make_async_copy(
            vals.at[pl.ds(e0, K), pl.ds(co, 32)], vbufs[b], vsems[b])

    for qj in range(2):
        co = 32 * (2 * c + qj)
        def zrow(r, _):
            sbuf[r, pl.ds(0, 16)] = zv
            sbuf[r, pl.ds(16, 16)] = zv
            return _
        lax.fori_loop(0, WR, zrow, None)

        def zchunk(kk, _):
            j = s + NS * kk
            @pl.when(j < nzch)
            def _():
                pltpu.sync_copy(sbuf, acc.at[pl.ds(j * WR, WR)])
            return _
        lax.fori_loop(0, kz, zchunk, None)
        plsc.subcore_barrier()

        _vload(0, 0, co).start()
        _vload(1, 1, co).start()

        def accum(kk, _):
            for b in (0, 1):
                k = 2 * kk + b
                _vload(k, b, co).wait()
                pltpu.async_copy(vbufs[b], acc.at[iball.at[k]],
                                 ssem, add=True).wait()
                @pl.when(kk < (CPT // 2 - 1))
                def _():
                    _vload(k + 2, b, co).start()
            return _
        lax.fori_loop(0, CPT // 2, accum, None)

        @pl.when(s < 2)
        def _():
            j = NS * CPT + s
            pltpu.sync_copy(idx2d.at[pl.ds(j, 1)], itail)
            pltpu.sync_copy(vals.at[pl.ds(j * K, K), pl.ds(co, 32)], vbuf0)
            pltpu.sync_copy(vbuf0, acc.at[itail.at[0]], add=True)
        plsc.subcore_barrier()

        wi = pbuf[0]
        we = pbuf[1]
        def wchunk(kk, _):
            j = s + NS * kk
            @pl.when(j < nzch)
            def _():
                r0 = j * WR
                pltpu.sync_copy(acc.at[pl.ds(r0, WR)], sbuf)
                pltpu.sync_copy(chn.at[pl.ds(r0, WR), pl.ds(co, 32)], cbuf)
                def wrow(r, _):
                    for h in (0, 16):
                        sl = pl.ds(h, 16)
                        sbuf[r, sl] = wi * cbuf[r, sl] + we * sbuf[r, sl]
                    return _
                lax.fori_loop(0, WR, wrow, None)
                pltpu.sync_copy(sbuf, out.at[pl.ds(r0, WR), pl.ds(co, 32)])
            return _
        lax.fori_loop(0, kz, wchunk, None)
        plsc.subcore_barrier()


@functools.partial(
    pl.kernel,
    out_type=jax.ShapeDtypeStruct((E, B), jnp.float32),
    mesh=_mesh,
    compiler_params=_params,
    scratch_types=[
        pltpu.VMEM((K, B), jnp.float32),
        pltpu.VMEM((K, B), jnp.float32),
        pltpu.VMEM((K, B), jnp.float32),
        pltpu.VMEM((K, B), jnp.float32),
        pltpu.VMEM((K, B), jnp.float32),
        pltpu.VMEM((K, B), jnp.float32),
        pltpu.VMEM((K, B), jnp.float32),
        pltpu.VMEM((CPW, K), jnp.int32),
        pltpu.VMEM((4, 16), jnp.float32),
        pltpu.SemaphoreType.DMA,
        pltpu.SemaphoreType.DMA,
        pltpu.SemaphoreType.DMA,
        pltpu.SemaphoreType.DMA,
        pltpu.SemaphoreType.DMA,
        pltpu.SemaphoreType.DMA,
        pltpu.SemaphoreType.DMA,
    ],
)
def _vstep(mv2c, mc2v, ellps, idx2d, par, out,
           abuf0, abuf1, bbuf0, bbuf1, gbuf0, gbuf1, obuf, iball, pbuf,
           asem0, asem1, bsem0, bsem1, gsem0, gsem1, osem):
    c = lax.axis_index("c")
    s = lax.axis_index("s")
    wid = s * NC + c
    start = jnp.minimum(wid * CPW, WSTART_MAX)
    pltpu.sync_copy(par, pbuf)
    pltpu.sync_copy(idx2d.at[pl.ds(start, CPW)], iball)
    omg = pbuf[0]
    g = pbuf[1]
    we = pbuf[2]
    abufs, bbufs, gbufs = (abuf0, abuf1), (bbuf0, bbuf1), (gbuf0, gbuf1)
    asems, bsems, gsems = (asem0, asem1), (bsem0, bsem1), (gsem0, gsem1)

    def _loads(k, b):
        e0 = (start + k) * K
        return (pltpu.make_async_copy(mv2c.at[pl.ds(e0, K)], abufs[b], asems[b]),
                pltpu.make_async_copy(mc2v.at[pl.ds(e0, K)], bbufs[b], bsems[b]),
                pltpu.make_async_copy(ellps.at[iball.at[k]], gbufs[b], gsems[b]))

    def _ostore(k):
        return pltpu.make_async_copy(obuf, out.at[pl.ds((start + k) * K, K)],
                                     osem)

    for d in _loads(0, 0):
        d.start()
    for d in _loads(1, 1):
        d.start()

    def step(kk, _):
        for b in (0, 1):
            k = 2 * kk + b
            for d in _loads(k, b):
                d.wait()
            if b == 0:
                @pl.when(kk > 0)
                def _():
                    _ostore(2 * kk - 1).wait()
            else:
                _ostore(2 * kk).wait()
            def row(r, _):
                for h in range(8):
                    sl = pl.ds(16 * h, 16)
                    obuf[r, sl] = (omg * abufs[b][r, sl]
                                   + g * (gbufs[b][r, sl]
                                          - we * bbufs[b][r, sl]))
                return _
            lax.fori_loop(0, K, row, None)
            _ostore(k).start()
            @pl.when(kk < (CPW // 2 - 1))
            def _():
                for d in _loads(k + 2, b):
                    d.start()
        return _
    lax.fori_loop(0, CPW // 2, step, None)
    _ostore(CPW - 1).wait()


@functools.partial(
    pl.kernel,
    out_type=(jax.ShapeDtypeStruct((M, B), jnp.float32),
              jax.ShapeDtypeStruct((M, B), jnp.float32)),
    mesh=_mesh,
    compiler_params=_params,
    scratch_types=[
        pltpu.VMEM_SHARED((M, 16), jnp.float32),
        pltpu.VMEM_SHARED((M, 16), jnp.float32),
        pltpu.VMEM((WR, 16), jnp.float32),
        pltpu.VMEM((K, 16), jnp.float32),
        pltpu.VMEM((K, 16), jnp.float32),
        pltpu.VMEM((K, 16), jnp.float32),
        pltpu.VMEM((K, 16), jnp.float32),
        pltpu.VMEM((K, 16), jnp.float32),
        pltpu.VMEM((K, 16), jnp.float32),
        pltpu.VMEM((CPT, K), jnp.int32),
        pltpu.VMEM((1, K), jnp.int32),
        pltpu.SemaphoreType.DMA,
        pltpu.SemaphoreType.DMA,
        pltpu.SemaphoreType.DMA,
        pltpu.SemaphoreType.DMA,
    ],
)
def _dual_seg(u, idx2d, st, sn, acct, accn, zbuf, ubuf0, ubuf1, tbuf0, tbuf1,
              nbuf0, nbuf1, iball, itail, usem0, usem1, tsem, nsem):
    c = lax.axis_index("c")
    s = lax.axis_index("s")
    pltpu.sync_copy(idx2d.at[pl.ds(s * CPT, CPT)], iball)
    zv = jnp.zeros((16,), jnp.float32)
    one = jnp.full((16,), 1.0, jnp.float32)
    nzch = M // WR
    kz = (nzch + NS - 1) // NS
    ubufs, tbufs, nbufs = (ubuf0, ubuf1), (tbuf0, tbuf1), (nbuf0, nbuf1)
    usems = (usem0, usem1)

    def zrow(r, _):
        zbuf[r, pl.ds(0, 16)] = zv
        return _
    lax.fori_loop(0, WR, zrow, None)

    def _uload(k, b, co):
        e0 = (s * CPT + k) * K
        return pltpu.make_async_copy(
            u.at[pl.ds(e0, K), pl.ds(co, 16)], ubufs[b], usems[b])

    for qj in range(4):
        co = 16 * (4 * c + qj)
        def zchunk(kk, _):
            j = s + NS * kk
            @pl.when(j < nzch)
            def _():
                pltpu.sync_copy(zbuf, acct.at[pl.ds(j * WR, WR)])
                pltpu.sync_copy(zbuf, accn.at[pl.ds(j * WR, WR)])
            return _
        lax.fori_loop(0, kz, zchunk, None)
        plsc.subcore_barrier()

        _uload(0, 0, co).start()
        _uload(1, 1, co).start()

        def accum(kk, _):
            for b in (0, 1):
                k = 2 * kk + b
                _uload(k, b, co).wait()
                def crow(r, _):
                    sl = pl.ds(0, 16)
                    x = ubufs[b][r, sl]
                    tbufs[b][r, sl] = -jnp.abs(x)
                    nbufs[b][r, sl] = jnp.where(x < 0, one, zv)
                    return _
                lax.fori_loop(0, K, crow, None)
                dt = pltpu.async_copy(tbufs[b], acct.at[iball.at[k]],
                                      tsem, add=True)
                dn = pltpu.async_copy(nbufs[b], accn.at[iball.at[k]],
                                      nsem, add=True)
                dt.wait()
                dn.wait()
                @pl.when(kk < (CPT // 2 - 1))
                def _():
                    _uload(k + 2, b, co).start()
            return _
        lax.fori_loop(0, CPT // 2, accum, None)

        @pl.when(s < 2)
        def _():
            j = NS * CPT + s
            pltpu.sync_copy(idx2d.at[pl.ds(j, 1)], itail)
            pltpu.sync_copy(u.at[pl.ds(j * K, K), pl.ds(co, 16)], ubuf0)
            def crow(r, _):
                sl = pl.ds(0, 16)
                x = ubuf0[r, sl]
                tbuf0[r, sl] = -jnp.abs(x)
                nbuf0[r, sl] = jnp.where(x < 0, one, zv)
                return _
            lax.fori_loop(0, K, crow, None)
            pltpu.sync_copy(tbuf0, acct.at[itail.at[0]], add=True)
            pltpu.sync_copy(nbuf0, accn.at[itail.at[0]], add=True)
        plsc.subcore_barrier()

        def wchunk(kk, _):
            j = s + NS * kk
            @pl.when(j < nzch)
            def _():
                r0 = j * WR
                pltpu.sync_copy(acct.at[pl.ds(r0, WR)],
                                st.at[pl.ds(r0, WR), pl.ds(co, 16)])
                pltpu.sync_copy(accn.at[pl.ds(r0, WR)],
                                sn.at[pl.ds(r0, WR), pl.ds(co, 16)])
            return _
        lax.fori_loop(0, kz, wchunk, None)
        plsc.subcore_barrier()


@functools.partial(
    pl.kernel,
    out_type=jax.ShapeDtypeStruct((E, B), jnp.float32),
    mesh=_mesh,
    compiler_params=_params,
    scratch_types=[
        pltpu.VMEM((K, B), jnp.float32),
        pltpu.VMEM((K, B), jnp.float32),
        pltpu.VMEM((K, B), jnp.float32),
        pltpu.VMEM((K, B), jnp.float32),
        pltpu.VMEM((K, B), jnp.float32),
        pltpu.VMEM((K, B), jnp.float32),
        pltpu.VMEM((K, B), jnp.float32),
        pltpu.VMEM((CPW, K), jnp.int32),
        pltpu.SemaphoreType.DMA,
        pltpu.SemaphoreType.DMA,
        pltpu.SemaphoreType.DMA,
        pltpu.SemaphoreType.DMA,
        pltpu.SemaphoreType.DMA,
        pltpu.SemaphoreType.DMA,
        pltpu.SemaphoreType.DMA,
    ],
)
def _hgather(u, st, sn, idx2d, out,
             ubuf0, ubuf1, tb0, tb1, nb0, nb1, obuf, iball,
             usemA, usemB, tsemA, tsemB, nsemA, nsemB, osem):
    c = lax.axis_index("c")
    s = lax.axis_index("s")
    wid = s * NC + c
    start = jnp.minimum(wid * CPW, WSTART_MAX)
    pltpu.sync_copy(idx2d.at[pl.ds(start, CPW)], iball)
    one = jnp.full((16,), 1.0, jnp.float32)
    zv = jnp.zeros((16,), jnp.float32)
    ubufs, tbs, nbs = (ubuf0, ubuf1), (tb0, tb1), (nb0, nb1)
    usems, tsems, nsems = (usemA, usemB), (tsemA, tsemB), (nsemA, nsemB)

    def _loads(k, b):
        e0 = (start + k) * K
        return (pltpu.make_async_copy(u.at[pl.ds(e0, K)], ubufs[b], usems[b]),
                pltpu.make_async_copy(st.at[iball.at[k]], tbs[b], tsems[b]),
                pltpu.make_async_copy(sn.at[iball.at[k]], nbs[b], nsems[b]))

    def _ostore(k):
        return pltpu.make_async_copy(obuf, out.at[pl.ds((start + k) * K, K)],
                                     osem)

    for d in _loads(0, 0):
        d.start()
    for d in _loads(1, 1):
        d.start()

    def step(kk, _):
        for b in (0, 1):
            k = 2 * kk + b
            for d in _loads(k, b):
                d.wait()
            if b == 0:
                @pl.when(kk > 0)
                def _():
                    _ostore(2 * kk - 1).wait()
            else:
                _ostore(2 * kk).wait()
            def row(r, _):
                for h in range(8):
                    sl = pl.ds(16 * h, 16)
                    uv = ubufs[b][r, sl]
                    tv = -jnp.abs(uv)
                    negv = jnp.where(uv < 0, one, zv)
                    amp = tbs[b][r, sl] - tv
                    cnt = nbs[b][r, sl] - negv
                    par = lax.shift_left(
                        lax.bitwise_and(cnt.astype(jnp.int32), 1), 31)
                    bits = lax.bitwise_or(
                        plsc.bitcast(jnp.abs(amp), jnp.int32), par)
                    obuf[r, sl] = plsc.bitcast(bits, jnp.float32)
                return _
            lax.fori_loop(0, K, row, None)
            _ostore(k).start()
            @pl.when(kk < (CPW // 2 - 1))
            def _():
                for d in _loads(k + 2, b):
                    d.start()
        return _
    lax.fori_loop(0, CPW // 2, step, None)
    _ostore(CPW - 1).wait()


_TCR = 1200


def _u_body(x_ref, o_ref):
    x = x_ref[...]
    lam = jnp.clip(x, -LLR_CLIP, LLR_CLIP)
    al = jnp.clip(jnp.abs(lam), LB, LLR_CLIP)
    t = jnp.log(jnp.tanh(al * 0.5))
    o_ref[...] = jnp.where(lam < 0, t, -t)


def _tc_u(x):
    return pl.pallas_call(
        _u_body,
        grid=(E // _TCR,),
        in_specs=[pl.BlockSpec((_TCR, B), lambda i: (i, 0))],
        out_specs=pl.BlockSpec((_TCR, B), lambda i: (i, 0)),
        out_shape=jax.ShapeDtypeStruct((E, B), jnp.float32),
    )(x)


def _h_body(g_ref, w_ref, m_ref, o_ref):
    g = g_ref[0, 0]
    w = w_ref[...]
    wb = lax.bitcast_convert_type(w, jnp.int32)
    sgn = jnp.where(wb < 0, -1.0, 1.0)
    y = jnp.exp(-jnp.abs(w)) * (1.0 - 1e-6)
    at = 0.5 * jnp.log((1.0 + y) / (1.0 - y))
    o_ref[...] = (1.0 - g) * m_ref[...] + g * (sgn * 2.0 * at)


def _tc_h(gamma, w, mc2v):
    return pl.pallas_call(
        _h_body,
        grid=(E // _TCR,),
        in_specs=[
            pl.BlockSpec(memory_space=pltpu.SMEM),
            pl.BlockSpec((_TCR, B), lambda i: (i, 0)),
            pl.BlockSpec((_TCR, B), lambda i: (i, 0)),
        ],
        out_specs=pl.BlockSpec((_TCR, B), lambda i: (i, 0)),
        out_shape=jax.ShapeDtypeStruct((E, B), jnp.float32),
    )(gamma, w, mc2v)


def kernel(chn_llr, msg_C2V, msg_V2C, gamma, edge_var, edge_chk, Wi, We):
    ev = jnp.reshape(edge_var.astype(jnp.int32), (NCH, K))
    ec = jnp.reshape(edge_chk.astype(jnp.int32), (NCH, K))
    wi = Wi[0]
    we = We[0]
    par1 = jnp.stack([jnp.full((16,), wi), jnp.full((16,), we)])
    par2 = jnp.stack([jnp.full((16,), 1.0 - gamma), jnp.full((16,), gamma),
                      jnp.full((16,), we), jnp.zeros((16,))])

    ellps = _seg_affine(msg_C2V, ev, chn_llr, par1)
    mv2c_new = _vstep(msg_V2C, msg_C2V, ellps, ev, par2)
    u = _tc_u(mv2c_new)
    st, sn = _dual_seg(u, ec)
    w = _hgather(u, st, sn, ec)
    mc2v_new = _tc_h(jnp.reshape(gamma, (1, 1)), w, msg_C2V)
    output = _seg_affine(mc2v_new, ev, chn_llr, par1)
    return (mc2v_new, mv2c_new, output)
